# baseline (speedup 1.0000x reference)
import functools

import jax
import jax.numpy as jnp
from jax import lax
from jax.experimental import pallas as pl
from jax.experimental.pallas import tpu as pltpu
from jax.experimental.pallas import tpu_sc as plsc

B = 4096
L = 50
D = 32
HALF = 16

NC = 2
NS = 16
NW = NC * NS

TOTAL = 3 * B
LP = 64
STEPS = TOTAL // NW
NBUF = 6
OCH = 128

NG = 250000
K1_RPW = 7816
K1_CH = 32
K1_NCH = 246


def _depad_body(table_hbm, t4_hbm, in_v, out_v, si0, si1, so0, so1):
    sins = (si0, si1)
    souts = (so0, so1)
    w = lax.axis_index("s") * NC + lax.axis_index("c")
    lo = w * K1_RPW
    hi = jnp.minimum(NG, lo + K1_RPW)

    def chunk_start(k):
        return jnp.minimum(lo + k * K1_CH, hi - K1_CH)

    def start_in(k, b):
        pltpu.make_async_copy(
            table_hbm.at[pl.ds(chunk_start(k) * 4, K1_CH * 4), :],
            in_v.at[b], sins[b]).start()

    start_in(0, 0)
    start_in(1, 1)

    def body(k, carry):
        for b in range(2):
            kk = k * 2 + b
            pltpu.make_async_copy(
                table_hbm.at[pl.ds(chunk_start(kk) * 4, K1_CH * 4), :],
                in_v.at[b], sins[b]).wait()

            @pl.when(kk >= 2)
            def _(b=b, kk=kk):
                pltpu.make_async_copy(
                    out_v.at[b], t4_hbm.at[pl.ds(chunk_start(kk - 2), K1_CH)],
                    souts[b]).wait()

            for r in range(K1_CH):
                for a in range(4):
                    for h in range(2):
                        out_v[b, r, pl.ds(a * D + h * HALF, HALF)] = (
                            in_v[b, r * 4 + a, pl.ds(h * HALF, HALF)])

            pltpu.make_async_copy(
                out_v.at[b], t4_hbm.at[pl.ds(chunk_start(kk), K1_CH)],
                souts[b]).start()

            @pl.when(kk + 2 < K1_NCH)
            def _(b=b, kk=kk):
                start_in(kk + 2, b)
        return carry

    lax.fori_loop(0, (K1_NCH + 1) // 2, body, 0)
    for b in range(2):
        pltpu.make_async_copy(
            out_v.at[b], t4_hbm.at[pl.ds(chunk_start(K1_NCH - 2 + b), K1_CH)],
            souts[b]).wait()


_sc_depad = functools.partial(
    pl.kernel,
    out_type=jax.ShapeDtypeStruct((NG, 4 * D), jnp.float32),
    mesh=plsc.VectorSubcoreMesh(core_axis_name="c", subcore_axis_name="s"),
    compiler_params=pltpu.CompilerParams(use_tc_tiling_on_sc=True),
    scratch_types=[
        pltpu.VMEM((2, K1_CH * 4, D), jnp.float32),
        pltpu.VMEM((2, K1_CH, 4 * D), jnp.float32),
        pltpu.SemaphoreType.DMA,
        pltpu.SemaphoreType.DMA,
        pltpu.SemaphoreType.DMA,
        pltpu.SemaphoreType.DMA,
    ],
)(_depad_body)


def _pool_body(gids_hbm, offs_hbm, t4_hbm, out_hbm, idx_v, offs_v, buf_v,
               out_v, *sems):
    w = lax.axis_index("s") * NC + lax.axis_index("c")
    base = w * (STEPS * LP)

    pltpu.sync_copy(gids_hbm.at[pl.ds(base, STEPS * LP)], idx_v)
    pltpu.sync_copy(offs_hbm.at[pl.ds(base, STEPS * LP)], offs_v)

    def start(t, b):
        pltpu.make_async_copy(
            t4_hbm.at[idx_v.at[pl.ds(t * LP, LP)]],
            buf_v.at[b], sems[b]).start()

    for b in range(NBUF):
        start(b, b)

    def outer(g, carry):
        for b in range(NBUF):
            t = g * NBUF + b
            pltpu.make_async_copy(
                t4_hbm.at[idx_v.at[pl.ds(t * LP, LP)]],
                buf_v.at[b], sems[b]).wait()

            ov = [offs_v[pl.ds(t * LP + blk, HALF)]
                  for blk in range(0, L, HALF)]

            def off(j):
                return pl.multiple_of(ov[j // HALF][j % HALF], D)

            o0 = off(0)
            o1 = off(1)
            a0 = buf_v[b, 0, pl.ds(o0, HALF)]
            a1 = buf_v[b, 0, pl.ds(o0 + HALF, HALF)]
            a2 = buf_v[b, 1, pl.ds(o1, HALF)]
            a3 = buf_v[b, 1, pl.ds(o1 + HALF, HALF)]
            for j in range(2, L, 2):
                oa = off(j)
                ob = off(j + 1)
                a0 = a0 + buf_v[b, j, pl.ds(oa, HALF)]
                a1 = a1 + buf_v[b, j, pl.ds(oa + HALF, HALF)]
                a2 = a2 + buf_v[b, j + 1, pl.ds(ob, HALF)]
                a3 = a3 + buf_v[b, j + 1, pl.ds(ob + HALF, HALF)]
            oo = pl.multiple_of((t & (OCH - 1)) * D, D)
            out_v[pl.ds(oo, HALF)] = a0 + a2
            out_v[pl.ds(oo + HALF, HALF)] = a1 + a3

            @pl.when((t & (OCH - 1)) == OCH - 1)
            def _(t=t):
                fo = pl.multiple_of((w * STEPS + t - (OCH - 1)) * D, OCH * D)
                pltpu.sync_copy(out_v, out_hbm.at[pl.ds(fo, OCH * D)])

            nxt = t + NBUF

            @pl.when(nxt < STEPS)
            def _(nxt=nxt, b=b):
                start(nxt, b)
        return carry

    lax.fori_loop(0, STEPS // NBUF, outer, 0)


_sc_pool = functools.partial(
    pl.kernel,
    out_type=jax.ShapeDtypeStruct((TOTAL * D,), jnp.float32),
    mesh=plsc.VectorSubcoreMesh(core_axis_name="c", subcore_axis_name="s"),
    compiler_params=pltpu.CompilerParams(use_tc_tiling_on_sc=True),
    scratch_types=[
        pltpu.VMEM((STEPS * LP,), jnp.int32),
        pltpu.VMEM((STEPS * LP,), jnp.int32),
        pltpu.VMEM((NBUF, LP, 4 * D), jnp.float32),
        pltpu.VMEM((OCH * D,), jnp.float32),
    ] + [pltpu.SemaphoreType.DMA] * NBUF,
)(_pool_body)


def _tc_loss_body(sums_ref, anchor_ref, loss_ref):
    a = sums_ref[0]
    p = sums_ref[1]
    n = sums_ref[2]
    an = a * lax.rsqrt(jnp.sum(a * a, axis=1, keepdims=True))
    pn = p * lax.rsqrt(jnp.sum(p * p, axis=1, keepdims=True))
    nn = n * lax.rsqrt(jnp.sum(n * n, axis=1, keepdims=True))
    anchor_ref[...] = an
    d_pos = jnp.sum((an - pn) ** 2, axis=1)
    d_neg = jnp.sum((an - nn) ** 2, axis=1)
    loss = jnp.mean(jnp.maximum(1.0 + d_pos - d_neg, 0.0))
    loss_ref[...] = jnp.reshape(loss, (1, 1))


_tc_loss = pl.pallas_call(
    _tc_loss_body,
    out_shape=(
        jax.ShapeDtypeStruct((B, D), jnp.float32),
        jax.ShapeDtypeStruct((1, 1), jnp.float32),
    ),
)


def kernel(anchor_input_ids, positive_input_ids, negative_input_ids,
           embedding_table):
    ids = jnp.concatenate(
        [anchor_input_ids, positive_input_ids, negative_input_ids[0]], axis=0)
    idsp = jnp.pad(ids, ((0, 0), (0, LP - L)))
    gids = (idsp >> 2).reshape(TOTAL * LP)
    offs = ((idsp & 3) * D).reshape(TOTAL * LP)
    t4 = _sc_depad(embedding_table)
    pooled = _sc_pool(gids, offs, t4)
    anchor, loss = _tc_loss(pooled.reshape(3, B, D))
    return anchor, loss[0, 0]


# speedup vs baseline: 12.7979x; 12.7979x over previous
import functools

import jax
import jax.numpy as jnp
from jax import lax
from jax.experimental import pallas as pl
from jax.experimental.pallas import tpu as pltpu
from jax.experimental.pallas import tpu_sc as plsc

B = 4096
L = 50
D = 32
HALF = 16

NC = 2
NS = 16
NW = NC * NS

TOTAL = 3 * B
S = 1
IDX = S * L
STEPS = TOTAL // (S * NW)
ROWS = STEPS * S
NBUF = 12


def _sc_pool_body(ids_hbm, table_hbm, out_hbm, idx_v, buf_v, out_v, *sems):
    w = lax.axis_index("s") * NC + lax.axis_index("c")
    base = w * STEPS

    pltpu.sync_copy(ids_hbm.at[pl.ds(base, STEPS)], idx_v)

    def start(t, b):
        pltpu.make_async_copy(
            table_hbm.at[idx_v.at[t]], buf_v.at[b], sems[b]).start()

    for b in range(NBUF):
        start(b, b)

    def outer(g, carry):
        for b in range(NBUF):
            t = g * NBUF + b
            pltpu.make_async_copy(
                table_hbm.at[idx_v.at[t]], buf_v.at[b], sems[b]).wait()

            for r in range(S):
                rb = r * L
                a0 = buf_v[b, rb + 0, pl.ds(0, HALF)]
                a1 = buf_v[b, rb + 0, pl.ds(HALF, HALF)]
                a2 = buf_v[b, rb + 1, pl.ds(0, HALF)]
                a3 = buf_v[b, rb + 1, pl.ds(HALF, HALF)]
                for j in range(2, L, 2):
                    a0 = a0 + buf_v[b, rb + j, pl.ds(0, HALF)]
                    a1 = a1 + buf_v[b, rb + j, pl.ds(HALF, HALF)]
                    a2 = a2 + buf_v[b, rb + j + 1, pl.ds(0, HALF)]
                    a3 = a3 + buf_v[b, rb + j + 1, pl.ds(HALF, HALF)]
                out_v[t * S + r, pl.ds(0, HALF)] = a0 + a2
                out_v[t * S + r, pl.ds(HALF, HALF)] = a1 + a3

            nxt = t + NBUF

            @pl.when(nxt < STEPS)
            def _(nxt=nxt, b=b):
                start(nxt, b)
        return carry

    lax.fori_loop(0, STEPS // NBUF, outer, 0)

    pltpu.sync_copy(out_v, out_hbm.at[pl.ds(w * ROWS, ROWS)])


_sc_pool = functools.partial(
    pl.kernel,
    out_type=jax.ShapeDtypeStruct((TOTAL, D), jnp.float32),
    mesh=plsc.VectorSubcoreMesh(core_axis_name="c", subcore_axis_name="s"),
    compiler_params=pltpu.CompilerParams(use_tc_tiling_on_sc=False),
    scratch_types=[
        pltpu.VMEM((STEPS, IDX), jnp.int32),
        pltpu.VMEM((NBUF, IDX, D), jnp.float32),
        pltpu.VMEM((ROWS, D), jnp.float32),
    ] + [pltpu.SemaphoreType.DMA] * NBUF,
)(_sc_pool_body)


def _tc_loss_body(sums_ref, anchor_ref, loss_ref):
    a = sums_ref[0]
    p = sums_ref[1]
    n = sums_ref[2]
    an = a * lax.rsqrt(jnp.sum(a * a, axis=1, keepdims=True))
    pn = p * lax.rsqrt(jnp.sum(p * p, axis=1, keepdims=True))
    nn = n * lax.rsqrt(jnp.sum(n * n, axis=1, keepdims=True))
    anchor_ref[...] = an
    d_pos = jnp.sum((an - pn) ** 2, axis=1)
    d_neg = jnp.sum((an - nn) ** 2, axis=1)
    loss = jnp.mean(jnp.maximum(1.0 + d_pos - d_neg, 0.0))
    loss_ref[...] = jnp.reshape(loss, (1, 1))


_tc_loss = pl.pallas_call(
    _tc_loss_body,
    out_shape=(
        jax.ShapeDtypeStruct((B, D), jnp.float32),
        jax.ShapeDtypeStruct((1, 1), jnp.float32),
    ),
)


def kernel(anchor_input_ids, positive_input_ids, negative_input_ids,
           embedding_table):
    ids = jnp.concatenate(
        [anchor_input_ids, positive_input_ids, negative_input_ids[0]], axis=0)
    pooled = _sc_pool(ids, embedding_table)
    anchor, loss = _tc_loss(pooled.reshape(3, B, D))
    return anchor, loss[0, 0]
